# Initial kernel scaffold; baseline (speedup 1.0000x reference)
import jax
import jax.numpy as jnp
from jax.experimental import pallas as pl


def kernel(x, edge_index, edge_attr, lin_e_w, lin_e_b, nn_w, nn_b, cls1_w, cls1_b, cls2_w, cls2_b):
    raise NotImplementedError("write your pallas kernel here")


# speedup vs baseline: 1.4469x; 1.4469x over previous
import jax
import jax.numpy as jnp
from jax import lax
from jax.experimental import pallas as pl
from jax.experimental.pallas import tpu as pltpu
from jax.experimental.pallas import tpu_sc as plsc

N_NODES = 10000
N_EDGES = 320000
D_FEAT = 128
D_HALF = D_FEAT // 2

NC = 2
NS = 16
NW = NC * NS

E_PER_W = N_EDGES // NS
CHUNK = 80
N_CHUNK = E_PER_W // CHUNK
ROW_SPLIT = 624


def _sc_aggregate_body(hfull_hbm, h2_hbm, ee_hbm, src_hbm, dst_hbm, out_hbm,
                       idx_s, idx_d, rows_v, msg_v, acc_sh, sem, sem_ee):
    c = lax.axis_index("c")
    s = lax.axis_index("s")

    r0 = pl.multiple_of(s * ROW_SPLIT, 8)
    last = N_NODES - (NS - 1) * ROW_SPLIT

    @pl.when(s < NS - 1)
    def _():
        pltpu.sync_copy(h2_hbm.at[c, pl.ds(r0, ROW_SPLIT)],
                        acc_sh.at[pl.ds(r0, ROW_SPLIT)])

    @pl.when(s == NS - 1)
    def _():
        r1 = (NS - 1) * ROW_SPLIT
        pltpu.sync_copy(h2_hbm.at[c, pl.ds(r1, last)],
                        acc_sh.at[pl.ds(r1, last)])

    plsc.subcore_barrier()
    coff = c * D_HALF

    def chunk_body(i, carry):
        e0 = s * E_PER_W + i * CHUNK
        pltpu.sync_copy(src_hbm.at[pl.ds(e0, CHUNK)], idx_s)
        pltpu.sync_copy(dst_hbm.at[pl.ds(e0, CHUNK)], idx_d)
        ee_cp = pltpu.async_copy(ee_hbm.at[c, pl.ds(e0, CHUNK)], msg_v, sem_ee)
        pltpu.async_copy(hfull_hbm.at[idx_s], rows_v, sem).wait()
        ee_cp.wait()

        def elt(e, _):
            for j in range(D_HALF // 16):
                sl = pl.ds(j * 16, 16)
                v = rows_v[e, pl.ds(coff + j * 16, 16)] + msg_v[e, sl]
                msg_v[e, sl] = jnp.maximum(v, 0.0)
            return 0
        lax.fori_loop(0, CHUNK, elt, 0, unroll=2)

        pltpu.sync_copy(msg_v, acc_sh.at[idx_d], add=True)
        return carry

    lax.fori_loop(0, N_CHUNK, chunk_body, 0)
    plsc.subcore_barrier()

    @pl.when(s < NS - 1)
    def _():
        pltpu.sync_copy(acc_sh.at[pl.ds(r0, ROW_SPLIT)],
                        out_hbm.at[c, pl.ds(r0, ROW_SPLIT)])

    @pl.when(s == NS - 1)
    def _():
        r1 = (NS - 1) * ROW_SPLIT
        pltpu.sync_copy(acc_sh.at[pl.ds(r1, last)],
                        out_hbm.at[c, pl.ds(r1, last)])


@jax.jit
def _sc_aggregate(hfull, h2, ee2, src1d, dst1d):
    return pl.kernel(
        _sc_aggregate_body,
        out_type=jax.ShapeDtypeStruct((NC, N_NODES, D_HALF), jnp.float32),
        mesh=plsc.VectorSubcoreMesh(core_axis_name="c", subcore_axis_name="s"),
        scratch_types=[
            pltpu.VMEM((CHUNK,), jnp.int32),
            pltpu.VMEM((CHUNK,), jnp.int32),
            pltpu.VMEM((CHUNK, D_FEAT), jnp.float32),
            pltpu.VMEM((CHUNK, D_HALF), jnp.float32),
            pltpu.VMEM_SHARED((N_NODES, D_HALF), jnp.float32),
            pltpu.SemaphoreType.DMA,
            pltpu.SemaphoreType.DMA,
        ],
    )(hfull, h2, ee2, src1d, dst1d)


def _ee_body(ea_ref, w_ref, b_ref, out_ref):
    y = jnp.dot(ea_ref[...], w_ref[...],
                preferred_element_type=jnp.float32) + b_ref[...]
    out_ref[0] = y[:, :D_HALF]
    out_ref[1] = y[:, D_HALF:]


@jax.jit
def _edge_emb(edge_attr, lin_e_wT, lin_e_b):
    blk = 6400
    grid = N_EDGES // blk
    return pl.pallas_call(
        _ee_body,
        grid=(grid,),
        in_specs=[
            pl.BlockSpec((blk, 16), lambda i: (i, 0)),
            pl.BlockSpec((16, D_FEAT), lambda i: (0, 0)),
            pl.BlockSpec((1, D_FEAT), lambda i: (0, 0)),
        ],
        out_specs=pl.BlockSpec((2, blk, D_HALF), lambda i: (0, i, 0)),
        out_shape=jax.ShapeDtypeStruct((2, N_EDGES, D_HALF), jnp.float32),
    )(edge_attr, lin_e_wT, lin_e_b)


def _layer_body(p_ref, wt_ref, wb_ref, b_ref, out_ref, out2_ref):
    y = (jnp.dot(p_ref[0], wt_ref[...], preferred_element_type=jnp.float32)
         + jnp.dot(p_ref[1], wb_ref[...], preferred_element_type=jnp.float32)
         + b_ref[...])
    y = jnp.maximum(y, 0.0)
    out_ref[...] = y
    out2_ref[0] = y[:, :D_HALF]
    out2_ref[1] = y[:, D_HALF:]


@jax.jit
def _tc_layer(p, w_top, w_bot, nn_b):
    blk = 2000
    grid = N_NODES // blk
    return pl.pallas_call(
        _layer_body,
        grid=(grid,),
        in_specs=[
            pl.BlockSpec((2, blk, D_HALF), lambda i: (0, i, 0)),
            pl.BlockSpec((D_HALF, D_FEAT), lambda i: (0, 0)),
            pl.BlockSpec((D_HALF, D_FEAT), lambda i: (0, 0)),
            pl.BlockSpec((1, D_FEAT), lambda i: (0, 0)),
        ],
        out_specs=[
            pl.BlockSpec((blk, D_FEAT), lambda i: (i, 0)),
            pl.BlockSpec((2, blk, D_HALF), lambda i: (0, i, 0)),
        ],
        out_shape=[
            jax.ShapeDtypeStruct((N_NODES, D_FEAT), jnp.float32),
            jax.ShapeDtypeStruct((2, N_NODES, D_HALF), jnp.float32),
        ],
    )(p, w_top, w_bot, nn_b)


def _final_body(p_ref, wt_ref, wb_ref, b_ref,
                w1_ref, b1_ref, w2_ref, b2_ref, out_ref):
    y = (jnp.dot(p_ref[0], wt_ref[...], preferred_element_type=jnp.float32)
         + jnp.dot(p_ref[1], wb_ref[...], preferred_element_type=jnp.float32)
         + b_ref[...])
    z = jnp.maximum(
        jnp.dot(y, w1_ref[...], preferred_element_type=jnp.float32)
        + b1_ref[...], 0.0)
    out_ref[...] = jnp.dot(z, w2_ref[...],
                           preferred_element_type=jnp.float32) + b2_ref[...]


@jax.jit
def _tc_final(p, w_top, w_bot, nn_b, cls1_wT, cls1_b, cls2_wT, cls2_b):
    blk = 2000
    grid = N_NODES // blk
    mid = cls1_wT.shape[1]
    out_ch = cls2_wT.shape[1]
    return pl.pallas_call(
        _final_body,
        grid=(grid,),
        in_specs=[
            pl.BlockSpec((2, blk, D_HALF), lambda i: (0, i, 0)),
            pl.BlockSpec((D_HALF, D_FEAT), lambda i: (0, 0)),
            pl.BlockSpec((D_HALF, D_FEAT), lambda i: (0, 0)),
            pl.BlockSpec((1, D_FEAT), lambda i: (0, 0)),
            pl.BlockSpec((D_FEAT, mid), lambda i: (0, 0)),
            pl.BlockSpec((1, mid), lambda i: (0, 0)),
            pl.BlockSpec((mid, out_ch), lambda i: (0, 0)),
            pl.BlockSpec((1, out_ch), lambda i: (0, 0)),
        ],
        out_specs=pl.BlockSpec((blk, out_ch), lambda i: (i, 0)),
        out_shape=jax.ShapeDtypeStruct((N_NODES, out_ch), jnp.float32),
    )(p, w_top, w_bot, nn_b, cls1_wT, cls1_b, cls2_wT, cls2_b)


def kernel(x, edge_index, edge_attr, lin_e_w, lin_e_b, nn_w, nn_b,
           cls1_w, cls1_b, cls2_w, cls2_b):
    src1d = edge_index[0]
    dst1d = edge_index[1]

    nn_wT = nn_w.T
    w_top = nn_wT[:D_HALF]
    w_bot = nn_wT[D_HALF:]
    nn_b2 = nn_b.reshape(1, -1)

    ee2 = _edge_emb(edge_attr, lin_e_w.T, lin_e_b.reshape(1, -1))
    x2 = jnp.stack([x[:, :D_HALF], x[:, D_HALF:]])

    p = _sc_aggregate(x, x2, ee2, src1d, dst1d)
    h1, h1_2 = _tc_layer(p, w_top, w_bot, nn_b2)

    p = _sc_aggregate(h1, h1_2, ee2, src1d, dst1d)
    pred = _tc_final(p, w_top, w_bot, nn_b2,
                     cls1_w.T, cls1_b.reshape(1, -1),
                     cls2_w.T, cls2_b.reshape(1, -1))
    return pred


# speedup vs baseline: 2.6619x; 1.8398x over previous
import jax
import jax.numpy as jnp
from jax import lax
from jax.experimental import pallas as pl
from jax.experimental.pallas import tpu as pltpu
from jax.experimental.pallas import tpu_sc as plsc

N_NODES = 10000
N_EDGES = 320000
D_FEAT = 128
D_HALF = D_FEAT // 2

NC = 2
NS = 16
NW = NC * NS

E_PER_W = N_EDGES // NS
CHUNK = 80
N_CHUNK = E_PER_W // CHUNK
ROW_SPLIT = 624


def _sc_aggregate_body(hfull_hbm, h2_hbm, ee_hbm, src_hbm, dst_hbm, out_hbm,
                       idx_s0, idx_d0, idx_s1, idx_d1,
                       idx_s2, idx_d2, idx_s3, idx_d3,
                       rows0, msg0, rows1, msg1,
                       acc_sh,
                       semI0, semI1, semI2, semI3,
                       semG0, semE0, semG1, semE1):
    c = lax.axis_index("c")
    s = lax.axis_index("s")

    r0 = pl.multiple_of(s * ROW_SPLIT, 8)
    last = N_NODES - (NS - 1) * ROW_SPLIT

    @pl.when(s < NS - 1)
    def _():
        pltpu.sync_copy(h2_hbm.at[c, pl.ds(r0, ROW_SPLIT)],
                        acc_sh.at[pl.ds(r0, ROW_SPLIT)])

    @pl.when(s == NS - 1)
    def _():
        r1 = (NS - 1) * ROW_SPLIT
        pltpu.sync_copy(h2_hbm.at[c, pl.ds(r1, last)],
                        acc_sh.at[pl.ds(r1, last)])

    plsc.subcore_barrier()
    coff = c * D_HALF
    base = s * E_PER_W

    IDX = [(idx_s0, idx_d0, semI0), (idx_s1, idx_d1, semI1),
           (idx_s2, idx_d2, semI2), (idx_s3, idx_d3, semI3)]
    DAT = [(rows0, msg0, semG0, semE0), (rows1, msg1, semG1, semE1)]

    def issue_idx(t, k):
        idx_s, idx_d, semI = IDX[k]
        e0 = pl.multiple_of(base + t * CHUNK, 8)
        pltpu.async_copy(src_hbm.at[pl.ds(e0, CHUNK)], idx_s, semI)
        pltpu.async_copy(dst_hbm.at[pl.ds(e0, CHUNK)], idx_d, semI)

    def issue_gather(t, d, k):
        idx_s, idx_d, semI = IDX[k]
        rows, msg, semG, semE = DAT[d]
        e0 = pl.multiple_of(base + t * CHUNK, 8)
        pltpu.make_async_copy(src_hbm.at[pl.ds(e0, CHUNK)], idx_s, semI).wait()
        pltpu.make_async_copy(dst_hbm.at[pl.ds(e0, CHUNK)], idx_d, semI).wait()
        pltpu.async_copy(hfull_hbm.at[idx_s], rows, semG)
        pltpu.async_copy(ee_hbm.at[c, pl.ds(e0, CHUNK)], msg, semE)

    def process(t, d, k):
        idx_s, idx_d, semI = IDX[k]
        rows, msg, semG, semE = DAT[d]
        e0 = pl.multiple_of(base + t * CHUNK, 8)
        pltpu.make_async_copy(hfull_hbm.at[idx_s], rows, semG).wait()
        pltpu.make_async_copy(ee_hbm.at[c, pl.ds(e0, CHUNK)], msg, semE).wait()

        def elt(e, _):
            for j in range(D_HALF // 16):
                sl = pl.ds(j * 16, 16)
                v = rows[e, pl.ds(coff + j * 16, 16)] + msg[e, sl]
                msg[e, sl] = jnp.maximum(v, 0.0)
            return 0
        lax.fori_loop(0, CHUNK, elt, 0, unroll=4)
        pltpu.sync_copy(msg, acc_sh.at[idx_d], add=True)

    issue_idx(0, 0)
    issue_idx(1, 1)
    issue_gather(0, 0, 0)

    def group(g, carry):
        t = g * 4
        for q in range(4):
            tq = t + q
            issue_gather(tq + 1, (q + 1) % 2, (q + 1) % 4)
            issue_idx(tq + 2, (q + 2) % 4)
            process(tq, q % 2, q % 4)
        return carry

    lax.fori_loop(0, (N_CHUNK - 2) // 4, group, 0)

    t_tail = N_CHUNK - 2
    issue_gather(t_tail + 1, 1, (t_tail + 1) % 4)
    process(t_tail, 0, t_tail % 4)
    process(t_tail + 1, 1, (t_tail + 1) % 4)

    plsc.subcore_barrier()

    @pl.when(s < NS - 1)
    def _():
        pltpu.sync_copy(acc_sh.at[pl.ds(r0, ROW_SPLIT)],
                        out_hbm.at[c, pl.ds(r0, ROW_SPLIT)])

    @pl.when(s == NS - 1)
    def _():
        r1 = (NS - 1) * ROW_SPLIT
        pltpu.sync_copy(acc_sh.at[pl.ds(r1, last)],
                        out_hbm.at[c, pl.ds(r1, last)])


@jax.jit
def _sc_aggregate(hfull, h2, ee2, src1d, dst1d):
    return pl.kernel(
        _sc_aggregate_body,
        out_type=jax.ShapeDtypeStruct((NC, N_NODES, D_HALF), jnp.float32),
        mesh=plsc.VectorSubcoreMesh(core_axis_name="c", subcore_axis_name="s"),
        scratch_types=(
            [pltpu.VMEM((CHUNK,), jnp.int32)] * 8
            + [pltpu.VMEM((CHUNK, D_FEAT), jnp.float32),
               pltpu.VMEM((CHUNK, D_HALF), jnp.float32)] * 2
            + [pltpu.VMEM_SHARED((N_NODES, D_HALF), jnp.float32)]
            + [pltpu.SemaphoreType.DMA] * 8
        ),
    )(hfull, h2, ee2, src1d, dst1d)


def _ee_body(ea_ref, w_ref, b_ref, out_ref):
    y = jnp.dot(ea_ref[...], w_ref[...],
                preferred_element_type=jnp.float32) + b_ref[...]
    out_ref[0] = y[:, :D_HALF]
    out_ref[1] = y[:, D_HALF:]


@jax.jit
def _edge_emb(edge_attr, lin_e_wT, lin_e_b):
    blk = 6400
    grid = N_EDGES // blk
    return pl.pallas_call(
        _ee_body,
        grid=(grid,),
        in_specs=[
            pl.BlockSpec((blk, 16), lambda i: (i, 0)),
            pl.BlockSpec((16, D_FEAT), lambda i: (0, 0)),
            pl.BlockSpec((1, D_FEAT), lambda i: (0, 0)),
        ],
        out_specs=pl.BlockSpec((2, blk, D_HALF), lambda i: (0, i, 0)),
        out_shape=jax.ShapeDtypeStruct((2, N_EDGES, D_HALF), jnp.float32),
    )(edge_attr, lin_e_wT, lin_e_b)


def _layer_body(p_ref, wt_ref, wb_ref, b_ref, out_ref, out2_ref):
    y = (jnp.dot(p_ref[0], wt_ref[...], preferred_element_type=jnp.float32)
         + jnp.dot(p_ref[1], wb_ref[...], preferred_element_type=jnp.float32)
         + b_ref[...])
    y = jnp.maximum(y, 0.0)
    out_ref[...] = y
    out2_ref[0] = y[:, :D_HALF]
    out2_ref[1] = y[:, D_HALF:]


@jax.jit
def _tc_layer(p, w_top, w_bot, nn_b):
    blk = 2000
    grid = N_NODES // blk
    return pl.pallas_call(
        _layer_body,
        grid=(grid,),
        in_specs=[
            pl.BlockSpec((2, blk, D_HALF), lambda i: (0, i, 0)),
            pl.BlockSpec((D_HALF, D_FEAT), lambda i: (0, 0)),
            pl.BlockSpec((D_HALF, D_FEAT), lambda i: (0, 0)),
            pl.BlockSpec((1, D_FEAT), lambda i: (0, 0)),
        ],
        out_specs=[
            pl.BlockSpec((blk, D_FEAT), lambda i: (i, 0)),
            pl.BlockSpec((2, blk, D_HALF), lambda i: (0, i, 0)),
        ],
        out_shape=[
            jax.ShapeDtypeStruct((N_NODES, D_FEAT), jnp.float32),
            jax.ShapeDtypeStruct((2, N_NODES, D_HALF), jnp.float32),
        ],
    )(p, w_top, w_bot, nn_b)


def _final_body(p_ref, wt_ref, wb_ref, b_ref,
                w1_ref, b1_ref, w2_ref, b2_ref, out_ref):
    y = (jnp.dot(p_ref[0], wt_ref[...], preferred_element_type=jnp.float32)
         + jnp.dot(p_ref[1], wb_ref[...], preferred_element_type=jnp.float32)
         + b_ref[...])
    z = jnp.maximum(
        jnp.dot(y, w1_ref[...], preferred_element_type=jnp.float32)
        + b1_ref[...], 0.0)
    out_ref[...] = jnp.dot(z, w2_ref[...],
                           preferred_element_type=jnp.float32) + b2_ref[...]


@jax.jit
def _tc_final(p, w_top, w_bot, nn_b, cls1_wT, cls1_b, cls2_wT, cls2_b):
    blk = 2000
    grid = N_NODES // blk
    mid = cls1_wT.shape[1]
    out_ch = cls2_wT.shape[1]
    return pl.pallas_call(
        _final_body,
        grid=(grid,),
        in_specs=[
            pl.BlockSpec((2, blk, D_HALF), lambda i: (0, i, 0)),
            pl.BlockSpec((D_HALF, D_FEAT), lambda i: (0, 0)),
            pl.BlockSpec((D_HALF, D_FEAT), lambda i: (0, 0)),
            pl.BlockSpec((1, D_FEAT), lambda i: (0, 0)),
            pl.BlockSpec((D_FEAT, mid), lambda i: (0, 0)),
            pl.BlockSpec((1, mid), lambda i: (0, 0)),
            pl.BlockSpec((mid, out_ch), lambda i: (0, 0)),
            pl.BlockSpec((1, out_ch), lambda i: (0, 0)),
        ],
        out_specs=pl.BlockSpec((blk, out_ch), lambda i: (i, 0)),
        out_shape=jax.ShapeDtypeStruct((N_NODES, out_ch), jnp.float32),
    )(p, w_top, w_bot, nn_b, cls1_wT, cls1_b, cls2_wT, cls2_b)


def kernel(x, edge_index, edge_attr, lin_e_w, lin_e_b, nn_w, nn_b,
           cls1_w, cls1_b, cls2_w, cls2_b):
    src1d = edge_index[0]
    dst1d = edge_index[1]

    nn_wT = nn_w.T
    w_top = nn_wT[:D_HALF]
    w_bot = nn_wT[D_HALF:]
    nn_b2 = nn_b.reshape(1, -1)

    ee2 = _edge_emb(edge_attr, lin_e_w.T, lin_e_b.reshape(1, -1))
    x2 = jnp.stack([x[:, :D_HALF], x[:, D_HALF:]])

    p = _sc_aggregate(x, x2, ee2, src1d, dst1d)
    h1, h1_2 = _tc_layer(p, w_top, w_bot, nn_b2)

    p = _sc_aggregate(h1, h1_2, ee2, src1d, dst1d)
    pred = _tc_final(p, w_top, w_bot, nn_b2,
                     cls1_w.T, cls1_b.reshape(1, -1),
                     cls2_w.T, cls2_b.reshape(1, -1))
    return pred


# speedup vs baseline: 2.6633x; 1.0005x over previous
import jax
import jax.numpy as jnp
from jax import lax
from jax.experimental import pallas as pl
from jax.experimental.pallas import tpu as pltpu
from jax.experimental.pallas import tpu_sc as plsc

N_NODES = 10000
N_EDGES = 320000
D_FEAT = 128
D_HALF = D_FEAT // 2

NC = 2
NS = 16
NW = NC * NS

CHUNK = 80
E_PER_W = N_EDGES // NS
N_CHUNK = E_PER_W // CHUNK
ROW_SPLIT = 624


def _sc_aggregate_body(hfull_hbm, h2_hbm, ee_hbm, src_hbm, dst_hbm, out_hbm,
                       is0, id0, is1, id1, is2, id2, is3, id3,
                       rows0, msg0, rows1, msg1,
                       acc_sh,
                       semI0, semI1, semI2, semI3,
                       semG0, semE0, semG1, semE1):
    c = lax.axis_index("c")
    s = lax.axis_index("s")

    r0 = pl.multiple_of(s * ROW_SPLIT, 8)
    last = N_NODES - (NS - 1) * ROW_SPLIT

    @pl.when(s < NS - 1)
    def _():
        pltpu.sync_copy(h2_hbm.at[c, pl.ds(r0, ROW_SPLIT)],
                        acc_sh.at[pl.ds(r0, ROW_SPLIT)])

    @pl.when(s == NS - 1)
    def _():
        r1 = (NS - 1) * ROW_SPLIT
        pltpu.sync_copy(h2_hbm.at[c, pl.ds(r1, last)],
                        acc_sh.at[pl.ds(r1, last)])

    plsc.subcore_barrier()
    coff = c * D_HALF
    base = s * E_PER_W

    IDX = [(is0, id0, semI0), (is1, id1, semI1),
           (is2, id2, semI2), (is3, id3, semI3)]
    DAT = [(rows0, msg0, semG0, semE0), (rows1, msg1, semG1, semE1)]

    def issue_idx(t, k):
        idx_s, idx_d, semI = IDX[k]
        e0 = pl.multiple_of(base + t * CHUNK, 8)
        pltpu.async_copy(src_hbm.at[pl.ds(e0, CHUNK)], idx_s, semI)
        pltpu.async_copy(dst_hbm.at[pl.ds(e0, CHUNK)], idx_d, semI)

    def issue_gather(t, d, k):
        idx_s, idx_d, semI = IDX[k]
        rows, msg, semG, semE = DAT[d]
        e0 = pl.multiple_of(base + t * CHUNK, 8)
        pltpu.make_async_copy(src_hbm.at[pl.ds(e0, CHUNK)], idx_s, semI).wait()
        pltpu.make_async_copy(dst_hbm.at[pl.ds(e0, CHUNK)], idx_d, semI).wait()
        pltpu.async_copy(hfull_hbm.at[idx_s], rows, semG)
        pltpu.async_copy(ee_hbm.at[c, pl.ds(e0, CHUNK)], msg, semE)

    def process(t, d, k):
        idx_s, idx_d, semI = IDX[k]
        rows, msg, semG, semE = DAT[d]
        e0 = pl.multiple_of(base + t * CHUNK, 8)
        pltpu.make_async_copy(hfull_hbm.at[idx_s], rows, semG).wait()
        pltpu.make_async_copy(ee_hbm.at[c, pl.ds(e0, CHUNK)], msg, semE).wait()

        def elt(e, _):
            for j in range(D_HALF // 16):
                sl = pl.ds(j * 16, 16)
                v = rows[e, pl.ds(coff + j * 16, 16)] + msg[e, sl]
                msg[e, sl] = jnp.maximum(v, 0.0)
            return 0
        lax.fori_loop(0, CHUNK, elt, 0, unroll=8)
        pltpu.sync_copy(msg, acc_sh.at[idx_d], add=True)

    issue_idx(0, 0)
    issue_idx(1, 1)
    issue_gather(0, 0, 0)

    def group(g, carry):
        t = g * 4
        for q in range(4):
            tq = t + q
            issue_gather(tq + 1, (q + 1) % 2, (q + 1) % 4)
            issue_idx(tq + 2, (q + 2) % 4)
            process(tq, q % 2, q % 4)
        return carry

    lax.fori_loop(0, (N_CHUNK - 2) // 4, group, 0)

    t_tail = N_CHUNK - 2
    issue_gather(t_tail + 1, 1, (t_tail + 1) % 4)
    process(t_tail, 0, t_tail % 4)
    process(t_tail + 1, 1, (t_tail + 1) % 4)

    plsc.subcore_barrier()

    @pl.when(s < NS - 1)
    def _():
        pltpu.sync_copy(acc_sh.at[pl.ds(r0, ROW_SPLIT)],
                        out_hbm.at[c, pl.ds(r0, ROW_SPLIT)])

    @pl.when(s == NS - 1)
    def _():
        r1 = (NS - 1) * ROW_SPLIT
        pltpu.sync_copy(acc_sh.at[pl.ds(r1, last)],
                        out_hbm.at[c, pl.ds(r1, last)])


@jax.jit
def _sc_aggregate(hfull, h2, ee2, src1d, dst1d):
    return pl.kernel(
        _sc_aggregate_body,
        out_type=jax.ShapeDtypeStruct((NC, N_NODES, D_HALF), jnp.float32),
        mesh=plsc.VectorSubcoreMesh(core_axis_name="c", subcore_axis_name="s"),
        scratch_types=(
            [pltpu.VMEM((CHUNK,), jnp.int32)] * 8
            + [pltpu.VMEM((CHUNK, D_FEAT), jnp.float32),
               pltpu.VMEM((CHUNK, D_HALF), jnp.float32)] * 2
            + [pltpu.VMEM_SHARED((N_NODES, D_HALF), jnp.float32)]
            + [pltpu.SemaphoreType.DMA] * 8
        ),
    )(hfull, h2, ee2, src1d, dst1d)


def _ee_body(ea_ref, w_ref, b_ref, out_ref):
    y = jnp.dot(ea_ref[...], w_ref[...],
                preferred_element_type=jnp.float32) + b_ref[...]
    out_ref[0] = y[:, :D_HALF]
    out_ref[1] = y[:, D_HALF:]


@jax.jit
def _edge_emb(edge_attr, lin_e_wT, lin_e_b):
    blk = 6400
    grid = N_EDGES // blk
    return pl.pallas_call(
        _ee_body,
        grid=(grid,),
        in_specs=[
            pl.BlockSpec((blk, 16), lambda i: (i, 0)),
            pl.BlockSpec((16, D_FEAT), lambda i: (0, 0)),
            pl.BlockSpec((1, D_FEAT), lambda i: (0, 0)),
        ],
        out_specs=pl.BlockSpec((2, blk, D_HALF), lambda i: (0, i, 0)),
        out_shape=jax.ShapeDtypeStruct((2, N_EDGES, D_HALF), jnp.float32),
    )(edge_attr, lin_e_wT, lin_e_b)


def _layer_body(p_ref, wt_ref, wb_ref, b_ref, out_ref, out2_ref):
    y = (jnp.dot(p_ref[0], wt_ref[...], preferred_element_type=jnp.float32)
         + jnp.dot(p_ref[1], wb_ref[...], preferred_element_type=jnp.float32)
         + b_ref[...])
    y = jnp.maximum(y, 0.0)
    out_ref[...] = y
    out2_ref[0] = y[:, :D_HALF]
    out2_ref[1] = y[:, D_HALF:]


@jax.jit
def _tc_layer(p, w_top, w_bot, nn_b):
    blk = 2000
    grid = N_NODES // blk
    return pl.pallas_call(
        _layer_body,
        grid=(grid,),
        in_specs=[
            pl.BlockSpec((2, blk, D_HALF), lambda i: (0, i, 0)),
            pl.BlockSpec((D_HALF, D_FEAT), lambda i: (0, 0)),
            pl.BlockSpec((D_HALF, D_FEAT), lambda i: (0, 0)),
            pl.BlockSpec((1, D_FEAT), lambda i: (0, 0)),
        ],
        out_specs=[
            pl.BlockSpec((blk, D_FEAT), lambda i: (i, 0)),
            pl.BlockSpec((2, blk, D_HALF), lambda i: (0, i, 0)),
        ],
        out_shape=[
            jax.ShapeDtypeStruct((N_NODES, D_FEAT), jnp.float32),
            jax.ShapeDtypeStruct((2, N_NODES, D_HALF), jnp.float32),
        ],
    )(p, w_top, w_bot, nn_b)


def _final_body(p_ref, wt_ref, wb_ref, b_ref,
                w1_ref, b1_ref, w2_ref, b2_ref, out_ref):
    y = (jnp.dot(p_ref[0], wt_ref[...], preferred_element_type=jnp.float32)
         + jnp.dot(p_ref[1], wb_ref[...], preferred_element_type=jnp.float32)
         + b_ref[...])
    z = jnp.maximum(
        jnp.dot(y, w1_ref[...], preferred_element_type=jnp.float32)
        + b1_ref[...], 0.0)
    out_ref[...] = jnp.dot(z, w2_ref[...],
                           preferred_element_type=jnp.float32) + b2_ref[...]


@jax.jit
def _tc_final(p, w_top, w_bot, nn_b, cls1_wT, cls1_b, cls2_wT, cls2_b):
    blk = 2000
    grid = N_NODES // blk
    mid = cls1_wT.shape[1]
    out_ch = cls2_wT.shape[1]
    return pl.pallas_call(
        _final_body,
        grid=(grid,),
        in_specs=[
            pl.BlockSpec((2, blk, D_HALF), lambda i: (0, i, 0)),
            pl.BlockSpec((D_HALF, D_FEAT), lambda i: (0, 0)),
            pl.BlockSpec((D_HALF, D_FEAT), lambda i: (0, 0)),
            pl.BlockSpec((1, D_FEAT), lambda i: (0, 0)),
            pl.BlockSpec((D_FEAT, mid), lambda i: (0, 0)),
            pl.BlockSpec((1, mid), lambda i: (0, 0)),
            pl.BlockSpec((mid, out_ch), lambda i: (0, 0)),
            pl.BlockSpec((1, out_ch), lambda i: (0, 0)),
        ],
        out_specs=pl.BlockSpec((blk, out_ch), lambda i: (i, 0)),
        out_shape=jax.ShapeDtypeStruct((N_NODES, out_ch), jnp.float32),
    )(p, w_top, w_bot, nn_b, cls1_wT, cls1_b, cls2_wT, cls2_b)


def kernel(x, edge_index, edge_attr, lin_e_w, lin_e_b, nn_w, nn_b,
           cls1_w, cls1_b, cls2_w, cls2_b):
    src1d = edge_index[0]
    dst1d = edge_index[1]

    nn_wT = nn_w.T
    w_top = nn_wT[:D_HALF]
    w_bot = nn_wT[D_HALF:]
    nn_b2 = nn_b.reshape(1, -1)

    ee2 = _edge_emb(edge_attr, lin_e_w.T, lin_e_b.reshape(1, -1))
    x2 = jnp.stack([x[:, :D_HALF], x[:, D_HALF:]])

    p = _sc_aggregate(x, x2, ee2, src1d, dst1d)
    h1, h1_2 = _tc_layer(p, w_top, w_bot, nn_b2)

    p = _sc_aggregate(h1, h1_2, ee2, src1d, dst1d)
    pred = _tc_final(p, w_top, w_bot, nn_b2,
                     cls1_w.T, cls1_b.reshape(1, -1),
                     cls2_w.T, cls2_b.reshape(1, -1))
    return pred
